# baseline (speedup 1.0000x reference)
import functools

import jax
import jax.numpy as jnp
from jax import lax
from jax.experimental import pallas as pl
from jax.experimental.pallas import tpu as pltpu
from jax.experimental.pallas import tpu_sc as plsc

_SC_PARAMS = pltpu.CompilerParams(
    use_tc_tiling_on_sc=True, needs_layout_passes=False)

_CW = 256
_NB = 4


def _relayout_lines(tabT, tail_lines, V, D, NC, NW, mesh):
    NBLK = V // _CW
    TAIL = V - NBLK * _CW
    LW = _CW // 4
    PER = -(-NBLK // NW)
    NLINES = -(-(V // 4) // 8) * 8

    @functools.partial(
        pl.kernel,
        mesh=mesh,
        out_type=jax.ShapeDtypeStruct((NLINES, 4 * D), jnp.float32),
        scratch_types=(
            [pltpu.VMEM((D, _CW), jnp.float32)] * _NB
            + [pltpu.VMEM((LW, 4 * D), jnp.float32)] * _NB
            + [pltpu.SemaphoreType.DMA] * (2 * _NB)
        ),
        compiler_params=_SC_PARAMS,
    )
    def run(tab_hbm, tail_hbm, lin_hbm, *sc):
        ins, ovs = sc[:_NB], sc[_NB:2 * _NB]
        sis, sos = sc[2 * _NB:3 * _NB], sc[3 * _NB:]
        w = lax.axis_index("s") * NC + lax.axis_index("c")
        n_w = jnp.minimum(PER, NBLK - w * PER)

        def in_copy(u, r):
            c = w * PER + u
            return pltpu.make_async_copy(
                tab_hbm.at[:, pl.ds(c * _CW, _CW)], ins[r], sis[r])

        def out_copy(u, r):
            c = w * PER + u
            return pltpu.make_async_copy(
                ovs[r], lin_hbm.at[pl.ds(c * LW, LW)], sos[r])

        def transpose_block(buf, ov):
            rows = [lax.iota(jnp.int32, 16) + 16 * h for h in range(2)]

            @plsc.parallel_loop(0, LW, unroll=16)
            def _(w2):
                cols = [jnp.full((16,), 4 * w2 + q, jnp.int32)
                        for q in range(4)]
                for g in range(8):
                    ov[w2, pl.ds(16 * g, 16)] = plsc.load_gather(
                        buf, [rows[g % 2], cols[g // 2]])

        for r in range(_NB):
            @pl.when(r < n_w)
            def _(r=r):
                in_copy(r, r).start()

        def body(m, carry):
            for r in range(_NB):
                u = _NB * m + r

                @pl.when(u < n_w)
                def _(u=u, r=r):
                    in_copy(u, r).wait()

                    @pl.when(u >= _NB)
                    def _():
                        out_copy(u - _NB, r).wait()

                    transpose_block(ins[r], ovs[r])
                    out_copy(u, r).start()

                    @pl.when(u + _NB < n_w)
                    def _():
                        in_copy(u + _NB, r).start()

            return carry

        lax.fori_loop(0, -(-PER // _NB), body, 0)

        for r in range(_NB):
            @pl.when(r < n_w)
            def _(r=r):
                out_copy(((n_w - 1 - r) // _NB) * _NB + r, r).wait()

        if TAIL:
            @pl.when(w == NW - 1)
            def _():
                pltpu.sync_copy(tail_hbm, ovs[0].at[pl.ds(0, TAIL // 4)])
                pltpu.sync_copy(ovs[0].at[pl.ds(0, TAIL // 4)],
                                lin_hbm.at[pl.ds(NBLK * LW, TAIL // 4)])

    return run(tabT, tail_lines)


def kernel(indices, table):
    B, T = indices.shape
    V, D = table.shape
    idxT = indices.T.astype(jnp.int32)
    tabT = table.T

    info = plsc.get_sparse_core_info()
    NC, NS = info.num_cores, info.num_subcores
    NW = NC * NS
    assert B == NW * 128 and D == 32 and T % _NB == 0
    tail = V % _CW
    assert tail % 32 == 0
    mesh = plsc.VectorSubcoreMesh(core_axis_name="c", subcore_axis_name="s")

    tail_lines = (table[V - tail:].reshape(tail // 4, 4 * D) if tail
                  else jnp.zeros((8, 4 * D), jnp.float32))
    lines = _relayout_lines(tabT, tail_lines, V, D, NC, NW, mesh)

    @functools.partial(
        pl.kernel,
        mesh=mesh,
        out_type=jax.ShapeDtypeStruct((T, D, B), jnp.float32),
        scratch_types=(
            [pltpu.VMEM((T, 128), jnp.int32)]
            + [pltpu.VMEM((128,), jnp.int32)] * (2 * _NB)
            + [pltpu.VMEM((128, 4 * D), jnp.float32)] * _NB
            + [pltpu.VMEM((D, 128), jnp.float32)] * _NB
            + [pltpu.SemaphoreType.DMA] * (2 * _NB)
        ),
        compiler_params=_SC_PARAMS,
    )
    def run(idx_hbm, lin_hbm, out_hbm, idx_v, *sc):
        lns, qcs = sc[:_NB], sc[_NB:2 * _NB]
        gs, ovs = sc[2 * _NB:3 * _NB], sc[3 * _NB:4 * _NB]
        sgs, sos = sc[4 * _NB:5 * _NB], sc[5 * _NB:]
        w = lax.axis_index("s") * NC + lax.axis_index("c")
        pltpu.sync_copy(idx_hbm.at[:, pl.ds(w * 128, 128)], idx_v)

        def prep(t, r):
            for g in range(8):
                v = idx_v[t, pl.ds(16 * g, 16)]
                qcs[r][pl.ds(16 * g, 16)] = (v & 3) * D
                lns[r][pl.ds(16 * g, 16)] = v >> 2

        def g_copy(r):
            return pltpu.make_async_copy(lin_hbm.at[lns[r]], gs[r], sgs[r])

        def out_copy(t, r):
            return pltpu.make_async_copy(
                ovs[r], out_hbm.at[t, :, pl.ds(w * 128, 128)], sos[r])

        def extract(r):
            rows = [lax.iota(jnp.int32, 16) + 16 * g for g in range(8)]
            qcv = [qcs[r][pl.ds(16 * g, 16)] for g in range(8)]

            @plsc.parallel_loop(0, D, unroll=8)
            def _(e):
                for g in range(8):
                    ovs[r][e, pl.ds(16 * g, 16)] = plsc.load_gather(
                        gs[r], [rows[g], qcv[g] + e])

        for r in range(_NB):
            prep(r, r)
            g_copy(r).start()

        def body(m, carry):
            for r in range(_NB):
                t = _NB * m + r
                g_copy(r).wait()

                @pl.when(t >= _NB)
                def _(t=t, r=r):
                    out_copy(t - _NB, r).wait()

                extract(r)
                out_copy(t, r).start()

                @pl.when(t + _NB < T)
                def _(t=t, r=r):
                    prep(t + _NB, r)
                    g_copy(r).start()

            return carry

        lax.fori_loop(0, T // _NB, body, 0)
        for r in range(_NB):
            out_copy(T - _NB + r, r).wait()

    outT = run(idxT, lines)
    return outT.transpose(2, 0, 1)


# speedup vs baseline: 1.2531x; 1.2531x over previous
import functools

import jax
import jax.numpy as jnp
from jax import lax
from jax.experimental import pallas as pl
from jax.experimental.pallas import tpu as pltpu
from jax.experimental.pallas import tpu_sc as plsc

_SC_PARAMS = pltpu.CompilerParams(
    use_tc_tiling_on_sc=True, needs_layout_passes=False)

_CW = 256
_NB = 4


def _relayout_lines(tabT, tail_lines, V, D, NC, NW, mesh):
    NBLK = V // _CW
    TAIL = V - NBLK * _CW
    LW = _CW // 4
    PER = -(-NBLK // NW)
    NLINES = -(-(V // 4) // 8) * 8

    @functools.partial(
        pl.kernel,
        mesh=mesh,
        out_type=jax.ShapeDtypeStruct((NLINES, 4 * D), jnp.float32),
        scratch_types=(
            [pltpu.VMEM((D, _CW), jnp.float32)] * _NB
            + [pltpu.VMEM((LW, 4 * D), jnp.float32)] * _NB
            + [pltpu.SemaphoreType.DMA] * (2 * _NB)
        ),
        compiler_params=_SC_PARAMS,
    )
    def run(tab_hbm, tail_hbm, lin_hbm, *sc):
        ins, ovs = sc[:_NB], sc[_NB:2 * _NB]
        sis, sos = sc[2 * _NB:3 * _NB], sc[3 * _NB:]
        w = lax.axis_index("s") * NC + lax.axis_index("c")
        n_w = jnp.minimum(PER, NBLK - w * PER)

        def in_copy(u, r):
            c = w * PER + u
            return pltpu.make_async_copy(
                tab_hbm.at[:, pl.ds(c * _CW, _CW)], ins[r], sis[r])

        def out_copy(u, r):
            c = w * PER + u
            return pltpu.make_async_copy(
                ovs[r], lin_hbm.at[pl.ds(c * LW, LW)], sos[r])

        def transpose_block(buf, ov):
            rows = [lax.iota(jnp.int32, 16) + 16 * h for h in range(2)]

            @plsc.parallel_loop(0, LW, unroll=16)
            def _(w2):
                cols = [jnp.full((16,), 4 * w2 + q, jnp.int32)
                        for q in range(4)]
                for g in range(8):
                    ov[w2, pl.ds(16 * g, 16)] = plsc.load_gather(
                        buf, [rows[g % 2], cols[g // 2]])

        for r in range(_NB):
            @pl.when(r < n_w)
            def _(r=r):
                in_copy(r, r).start()

        def body(m, carry):
            for r in range(_NB):
                u = _NB * m + r

                @pl.when(u < n_w)
                def _(u=u, r=r):
                    in_copy(u, r).wait()

                    @pl.when(u >= _NB)
                    def _():
                        out_copy(u - _NB, r).wait()

                    transpose_block(ins[r], ovs[r])
                    out_copy(u, r).start()

                    @pl.when(u + _NB < n_w)
                    def _():
                        in_copy(u + _NB, r).start()

            return carry

        lax.fori_loop(0, -(-PER // _NB), body, 0)

        for r in range(_NB):
            @pl.when(r < n_w)
            def _(r=r):
                out_copy(((n_w - 1 - r) // _NB) * _NB + r, r).wait()

        if TAIL:
            @pl.when(w == NW - 1)
            def _():
                pltpu.sync_copy(tail_hbm, ovs[0].at[pl.ds(0, TAIL // 4)])
                pltpu.sync_copy(ovs[0].at[pl.ds(0, TAIL // 4)],
                                lin_hbm.at[pl.ds(NBLK * LW, TAIL // 4)])

    return run(tabT, tail_lines)


def kernel(indices, table):
    B, T = indices.shape
    V, D = table.shape
    idxT = indices.T.astype(jnp.int32)
    tabT = table.T

    info = plsc.get_sparse_core_info()
    NC, NS = info.num_cores, info.num_subcores
    NW = NC * NS
    assert B == NW * 128 and D == 32 and T % _NB == 0
    tail = V % _CW
    assert tail % 32 == 0
    mesh = plsc.VectorSubcoreMesh(core_axis_name="c", subcore_axis_name="s")

    tail_lines = (table[V - tail:].reshape(tail // 4, 4 * D) if tail
                  else jnp.zeros((8, 4 * D), jnp.float32))
    lines = _relayout_lines(tabT, tail_lines, V, D, NC, NW, mesh)

    @functools.partial(
        pl.kernel,
        mesh=mesh,
        out_type=jax.ShapeDtypeStruct((T, D, B), jnp.float32),
        scratch_types=(
            [pltpu.VMEM((T, 128), jnp.int32)]
            + [pltpu.VMEM((128,), jnp.int32)] * (2 * _NB)
            + [pltpu.VMEM((128, 4 * D), jnp.float32)] * _NB
            + [pltpu.VMEM((D, 128), jnp.float32)] * _NB
            + [pltpu.SemaphoreType.DMA] * (2 * _NB)
        ),
        compiler_params=_SC_PARAMS,
    )
    def run(idx_hbm, lin_hbm, out_hbm, idx_v, *sc):
        lns, qcs = sc[:_NB], sc[_NB:2 * _NB]
        gs, ovs = sc[2 * _NB:3 * _NB], sc[3 * _NB:4 * _NB]
        sgs, sos = sc[4 * _NB:5 * _NB], sc[5 * _NB:]
        w = lax.axis_index("s") * NC + lax.axis_index("c")
        pltpu.sync_copy(idx_hbm.at[:, pl.ds(w * 128, 128)], idx_v)

        def prep(t, r):
            for g in range(8):
                v = idx_v[t, pl.ds(16 * g, 16)]
                qcs[r][pl.ds(16 * g, 16)] = (v & 3) * D
                lns[r][pl.ds(16 * g, 16)] = v >> 2

        def g_copy(r):
            return pltpu.make_async_copy(lin_hbm.at[lns[r]], gs[r], sgs[r])

        def out_copy(t, r):
            return pltpu.make_async_copy(
                ovs[r], out_hbm.at[t, :, pl.ds(w * 128, 128)], sos[r])

        def extract(r):
            iota = lax.iota(jnp.int32, 16)
            pos = [iota + 16 * g for g in range(8)]
            qcv = [qcs[r][pl.ds(16 * g, 16)] for g in range(8)]

            @plsc.parallel_loop(0, 16, unroll=4)
            def _(e0):
                em = (iota + e0) & 15
                for h in range(D // 16):
                    emh = em + 16 * h
                    for g in range(8):
                        v = plsc.load_gather(gs[r], [pos[g], qcv[g] + emh])
                        plsc.store_scatter(ovs[r], [emh, pos[g]], v)

        for r in range(_NB):
            prep(r, r)
            g_copy(r).start()

        def body(m, carry):
            for r in range(_NB):
                t = _NB * m + r
                g_copy(r).wait()

                @pl.when(t >= _NB)
                def _(t=t, r=r):
                    out_copy(t - _NB, r).wait()

                extract(r)
                out_copy(t, r).start()

                @pl.when(t + _NB < T)
                def _(t=t, r=r):
                    prep(t + _NB, r)
                    g_copy(r).start()

            return carry

        lax.fori_loop(0, T // _NB, body, 0)
        for r in range(_NB):
            out_copy(T - _NB + r, r).wait()

    outT = run(idxT, lines)
    return outT.transpose(2, 0, 1)


# speedup vs baseline: 1.5967x; 1.2742x over previous
import functools

import jax
import jax.numpy as jnp
from jax import lax
from jax.experimental import pallas as pl
from jax.experimental.pallas import tpu as pltpu
from jax.experimental.pallas import tpu_sc as plsc

_SC_PARAMS = pltpu.CompilerParams(
    use_tc_tiling_on_sc=True, needs_layout_passes=False)

_CW = 256
_NB = 4


def _relayout_lines(tabT, tail_lines, V, D, NC, NW, mesh):
    NBLK = V // _CW
    TAIL = V - NBLK * _CW
    LW = _CW // 4
    PER = -(-NBLK // NW)
    NLINES = -(-(V // 4) // 8) * 8

    @functools.partial(
        pl.kernel,
        mesh=mesh,
        out_type=jax.ShapeDtypeStruct((NLINES, 4 * D), jnp.float32),
        scratch_types=(
            [pltpu.VMEM((D, _CW), jnp.float32)] * _NB
            + [pltpu.VMEM((LW, 4 * D), jnp.float32)] * _NB
            + [pltpu.SemaphoreType.DMA] * (2 * _NB)
        ),
        compiler_params=_SC_PARAMS,
    )
    def run(tab_hbm, tail_hbm, lin_hbm, *sc):
        ins, ovs = sc[:_NB], sc[_NB:2 * _NB]
        sis, sos = sc[2 * _NB:3 * _NB], sc[3 * _NB:]
        w = lax.axis_index("s") * NC + lax.axis_index("c")
        n_w = jnp.minimum(PER, NBLK - w * PER)

        def in_copy(u, r):
            c = w * PER + u
            return pltpu.make_async_copy(
                tab_hbm.at[:, pl.ds(c * _CW, _CW)], ins[r], sis[r])

        def out_copy(u, r):
            c = w * PER + u
            return pltpu.make_async_copy(
                ovs[r], lin_hbm.at[pl.ds(c * LW, LW)], sos[r])

        def transpose_block(buf, ov):
            iota = lax.iota(jnp.int32, 16)

            @plsc.parallel_loop(0, _CW // 16, unroll=4)
            def _(k):
                cvec = iota + 16 * k
                rv = cvec >> 2
                cbase = (cvec & 3) * D
                for h in range(D // 16):
                    for e0 in range(16):
                        em = ((iota + e0) & 15) + 16 * h
                        v = plsc.load_gather(buf, [em, cvec])
                        plsc.store_scatter(ov, [rv, cbase + em], v)

        for r in range(_NB):
            @pl.when(r < n_w)
            def _(r=r):
                in_copy(r, r).start()

        def body(m, carry):
            for r in range(_NB):
                u = _NB * m + r

                @pl.when(u < n_w)
                def _(u=u, r=r):
                    in_copy(u, r).wait()

                    @pl.when(u >= _NB)
                    def _():
                        out_copy(u - _NB, r).wait()

                    transpose_block(ins[r], ovs[r])
                    out_copy(u, r).start()

                    @pl.when(u + _NB < n_w)
                    def _():
                        in_copy(u + _NB, r).start()

            return carry

        lax.fori_loop(0, -(-PER // _NB), body, 0)

        for r in range(_NB):
            @pl.when(r < n_w)
            def _(r=r):
                out_copy(((n_w - 1 - r) // _NB) * _NB + r, r).wait()

        if TAIL:
            @pl.when(w == NW - 1)
            def _():
                pltpu.sync_copy(tail_hbm, ovs[0].at[pl.ds(0, TAIL // 4)])
                pltpu.sync_copy(ovs[0].at[pl.ds(0, TAIL // 4)],
                                lin_hbm.at[pl.ds(NBLK * LW, TAIL // 4)])

    return run(tabT, tail_lines)


def kernel(indices, table):
    B, T = indices.shape
    V, D = table.shape
    idxT = indices.T.astype(jnp.int32)
    tabT = table.T

    info = plsc.get_sparse_core_info()
    NC, NS = info.num_cores, info.num_subcores
    NW = NC * NS
    assert B == NW * 128 and D == 32 and T % _NB == 0
    tail = V % _CW
    assert tail % 32 == 0
    mesh = plsc.VectorSubcoreMesh(core_axis_name="c", subcore_axis_name="s")

    tail_lines = (table[V - tail:].reshape(tail // 4, 4 * D) if tail
                  else jnp.zeros((8, 4 * D), jnp.float32))
    lines = _relayout_lines(tabT, tail_lines, V, D, NC, NW, mesh)

    @functools.partial(
        pl.kernel,
        mesh=mesh,
        out_type=jax.ShapeDtypeStruct((T, D, B), jnp.float32),
        scratch_types=(
            [pltpu.VMEM((T, 128), jnp.int32)]
            + [pltpu.VMEM((128,), jnp.int32)] * (2 * _NB)
            + [pltpu.VMEM((128, 4 * D), jnp.float32)] * _NB
            + [pltpu.VMEM((D, 128), jnp.float32)] * _NB
            + [pltpu.SemaphoreType.DMA] * (2 * _NB)
        ),
        compiler_params=_SC_PARAMS,
    )
    def run(idx_hbm, lin_hbm, out_hbm, idx_v, *sc):
        lns, qcs = sc[:_NB], sc[_NB:2 * _NB]
        gs, ovs = sc[2 * _NB:3 * _NB], sc[3 * _NB:4 * _NB]
        sgs, sos = sc[4 * _NB:5 * _NB], sc[5 * _NB:]
        w = lax.axis_index("s") * NC + lax.axis_index("c")
        pltpu.sync_copy(idx_hbm.at[:, pl.ds(w * 128, 128)], idx_v)

        def prep(t, r):
            for g in range(8):
                v = idx_v[t, pl.ds(16 * g, 16)]
                qcs[r][pl.ds(16 * g, 16)] = (v & 3) * D
                lns[r][pl.ds(16 * g, 16)] = v >> 2

        def g_copy(r):
            return pltpu.make_async_copy(lin_hbm.at[lns[r]], gs[r], sgs[r])

        def out_copy(t, r):
            return pltpu.make_async_copy(
                ovs[r], out_hbm.at[t, :, pl.ds(w * 128, 128)], sos[r])

        def extract(r):
            iota = lax.iota(jnp.int32, 16)
            pos = [iota + 16 * g for g in range(8)]
            qcv = [qcs[r][pl.ds(16 * g, 16)] for g in range(8)]

            @plsc.parallel_loop(0, 16, unroll=4)
            def _(e0):
                em = (iota + e0) & 15
                for h in range(D // 16):
                    emh = em + 16 * h
                    for g in range(8):
                        v = plsc.load_gather(gs[r], [pos[g], qcv[g] + emh])
                        plsc.store_scatter(ovs[r], [emh, pos[g]], v)

        for r in range(_NB):
            prep(r, r)
            g_copy(r).start()

        def body(m, carry):
            for r in range(_NB):
                t = _NB * m + r
                g_copy(r).wait()

                @pl.when(t >= _NB)
                def _(t=t, r=r):
                    out_copy(t - _NB, r).wait()

                extract(r)
                out_copy(t, r).start()

                @pl.when(t + _NB < T)
                def _(t=t, r=r):
                    prep(t + _NB, r)
                    g_copy(r).start()

            return carry

        lax.fori_loop(0, T // _NB, body, 0)
        for r in range(_NB):
            out_copy(T - _NB + r, r).wait()

    outT = run(idxT, lines)
    return outT.transpose(2, 0, 1)


# speedup vs baseline: 2.2950x; 1.4374x over previous
import functools

import jax
import jax.numpy as jnp
from jax import lax
from jax.experimental import pallas as pl
from jax.experimental.pallas import tpu as pltpu
from jax.experimental.pallas import tpu_sc as plsc

_SC_PARAMS = pltpu.CompilerParams(
    use_tc_tiling_on_sc=True, needs_layout_passes=False)

_CW = 256
_NB = 4


def _relayout_lines(tabT, tail_lines, V, D, NC, NW, mesh):
    NBLK = V // _CW
    TAIL = V - NBLK * _CW
    LW = _CW // 4
    PER = -(-NBLK // NW)
    NLINES = -(-(V // 4) // 8) * 8

    @functools.partial(
        pl.kernel,
        mesh=mesh,
        out_type=jax.ShapeDtypeStruct((NLINES, 4 * D), jnp.float32),
        scratch_types=(
            [pltpu.VMEM((D, _CW), jnp.float32)] * _NB
            + [pltpu.VMEM((LW, 4 * D), jnp.float32)] * _NB
            + [pltpu.SemaphoreType.DMA] * (2 * _NB)
        ),
        compiler_params=_SC_PARAMS,
    )
    def run(tab_hbm, tail_hbm, lin_hbm, *sc):
        ins, ovs = sc[:_NB], sc[_NB:2 * _NB]
        sis, sos = sc[2 * _NB:3 * _NB], sc[3 * _NB:]
        w = lax.axis_index("s") * NC + lax.axis_index("c")
        n_w = jnp.minimum(PER, NBLK - w * PER)

        def in_copy(u, r):
            c = w * PER + u
            return pltpu.make_async_copy(
                tab_hbm.at[:, pl.ds(c * _CW, _CW)], ins[r], sis[r])

        def out_copy(u, r):
            c = w * PER + u
            return pltpu.make_async_copy(
                ovs[r], lin_hbm.at[pl.ds(c * LW, LW)], sos[r])

        def transpose_block(buf, ov):
            iota = lax.iota(jnp.int32, 16)

            @plsc.parallel_loop(0, _CW // 16, unroll=8)
            def _(k):
                cvec = iota + 16 * k
                rv = cvec >> 2
                cbase = (cvec & 3) * D
                for h in range(D // 16):
                    for e0 in range(16):
                        em = ((iota + e0) & 15) + 16 * h
                        v = plsc.load_gather(buf, [em, cvec])
                        plsc.store_scatter(ov, [rv, cbase + em], v)

        for r in range(_NB):
            @pl.when(r < n_w)
            def _(r=r):
                in_copy(r, r).start()

        def body(m, carry):
            for r in range(_NB):
                u = _NB * m + r

                @pl.when(u < n_w)
                def _(u=u, r=r):
                    in_copy(u, r).wait()

                    @pl.when(u >= _NB)
                    def _():
                        out_copy(u - _NB, r).wait()

                    transpose_block(ins[r], ovs[r])
                    out_copy(u, r).start()

                    @pl.when(u + _NB < n_w)
                    def _():
                        in_copy(u + _NB, r).start()

            return carry

        lax.fori_loop(0, -(-PER // _NB), body, 0)

        for r in range(_NB):
            @pl.when(r < n_w)
            def _(r=r):
                out_copy(((n_w - 1 - r) // _NB) * _NB + r, r).wait()

        if TAIL:
            @pl.when(w == NW - 1)
            def _():
                pltpu.sync_copy(tail_hbm, ovs[0].at[pl.ds(0, TAIL // 4)])
                pltpu.sync_copy(ovs[0].at[pl.ds(0, TAIL // 4)],
                                lin_hbm.at[pl.ds(NBLK * LW, TAIL // 4)])

    return run(tabT, tail_lines)


def kernel(indices, table):
    B, T = indices.shape
    V, D = table.shape
    idxT = indices.T.astype(jnp.int32)
    tabT = table.T

    info = plsc.get_sparse_core_info()
    NC, NS = info.num_cores, info.num_subcores
    NW = NC * NS
    assert B == NW * 128 and D == 32 and T % _NB == 0
    tail = V % _CW
    assert tail % 32 == 0
    mesh = plsc.VectorSubcoreMesh(core_axis_name="c", subcore_axis_name="s")

    tail_lines = (table[V - tail:].reshape(tail // 4, 4 * D) if tail
                  else jnp.zeros((8, 4 * D), jnp.float32))
    lines = _relayout_lines(tabT, tail_lines, V, D, NC, NW, mesh)

    @functools.partial(
        pl.kernel,
        mesh=mesh,
        out_type=jax.ShapeDtypeStruct((T, D, B), jnp.float32),
        scratch_types=(
            [pltpu.VMEM((T, 128), jnp.int32)]
            + [pltpu.VMEM((128,), jnp.int32)] * (2 * _NB)
            + [pltpu.VMEM((128, 4 * D), jnp.float32)] * _NB
            + [pltpu.VMEM((D, 128), jnp.float32)] * _NB
            + [pltpu.SemaphoreType.DMA] * (2 * _NB)
        ),
        compiler_params=_SC_PARAMS,
    )
    def run(idx_hbm, lin_hbm, out_hbm, idx_v, *sc):
        lns, qcs = sc[:_NB], sc[_NB:2 * _NB]
        gs, ovs = sc[2 * _NB:3 * _NB], sc[3 * _NB:4 * _NB]
        sgs, sos = sc[4 * _NB:5 * _NB], sc[5 * _NB:]
        w = lax.axis_index("s") * NC + lax.axis_index("c")
        pltpu.sync_copy(idx_hbm.at[:, pl.ds(w * 128, 128)], idx_v)

        def prep(t, r):
            for g in range(8):
                v = idx_v[t, pl.ds(16 * g, 16)]
                qcs[r][pl.ds(16 * g, 16)] = (v & 3) * D
                lns[r][pl.ds(16 * g, 16)] = v >> 2

        def g_copy(r):
            return pltpu.make_async_copy(lin_hbm.at[lns[r]], gs[r], sgs[r])

        def out_copy(t, r):
            return pltpu.make_async_copy(
                ovs[r], out_hbm.at[t, :, pl.ds(w * 128, 128)], sos[r])

        def extract(r):
            iota = lax.iota(jnp.int32, 16)
            pos = [iota + 16 * g for g in range(8)]
            qcv = [qcs[r][pl.ds(16 * g, 16)] for g in range(8)]

            @plsc.parallel_loop(0, 16, unroll=4)
            def _(e0):
                em = (iota + e0) & 15
                for h in range(D // 16):
                    emh = em + 16 * h
                    for g in range(8):
                        v = plsc.load_gather(gs[r], [pos[g], qcv[g] + emh])
                        plsc.store_scatter(ovs[r], [emh, pos[g]], v)

        for r in range(_NB):
            prep(r, r)
            g_copy(r).start()

        def body(m, carry):
            for r in range(_NB):
                t = _NB * m + r
                g_copy(r).wait()

                @pl.when(t >= _NB)
                def _(t=t, r=r):
                    out_copy(t - _NB, r).wait()

                extract(r)
                out_copy(t, r).start()

                @pl.when(t + _NB < T)
                def _(t=t, r=r):
                    prep(t + _NB, r)
                    g_copy(r).start()

            return carry

        lax.fori_loop(0, T // _NB, body, 0)
        for r in range(_NB):
            out_copy(T - _NB + r, r).wait()

    outT = run(idxT, lines)
    return outT.transpose(2, 0, 1)
